# Initial kernel scaffold; baseline (speedup 1.0000x reference)
import jax
import jax.numpy as jnp
from jax.experimental import pallas as pl


def kernel(input, masked_values):
    raise NotImplementedError("write your pallas kernel here")


# speedup vs baseline: 1.4997x; 1.4997x over previous
import jax
import jax.numpy as jnp
from jax.experimental import pallas as pl

_BR = 1024


def _mask_body(x_ref, mv_ref, o_ref):
    x = x_ref[...]
    mv = mv_ref[0, 0, :]
    col = jax.lax.broadcasted_iota(jnp.int32, x.shape, 1)
    o_ref[...] = jnp.where(col == mv[:, None], jnp.float32(-1.0), x)


def kernel(input, masked_values):
    B, C = input.shape
    grid = (B // _BR,)
    mv3 = masked_values.reshape(grid[0], 1, _BR)
    return pl.pallas_call(
        _mask_body,
        grid=grid,
        in_specs=[
            pl.BlockSpec((_BR, C), lambda i: (i, 0)),
            pl.BlockSpec((1, 1, _BR), lambda i: (i, 0, 0)),
        ],
        out_specs=pl.BlockSpec((_BR, C), lambda i: (i, 0)),
        out_shape=jax.ShapeDtypeStruct((B, C), input.dtype),
    )(input, mv3)


# speedup vs baseline: 1.5126x; 1.0086x over previous
import jax
import jax.numpy as jnp
from jax.experimental import pallas as pl

_BR = 2048


def _mask_body(x_ref, mv_ref, o_ref):
    x = x_ref[...]
    mv = mv_ref[0, 0, :]
    col = jax.lax.broadcasted_iota(jnp.int32, x.shape, 1)
    o_ref[...] = jnp.where(col == mv[:, None], jnp.float32(-1.0), x)


def kernel(input, masked_values):
    B, C = input.shape
    grid = (B // _BR,)
    mv3 = masked_values.reshape(grid[0], 1, _BR)
    return pl.pallas_call(
        _mask_body,
        grid=grid,
        in_specs=[
            pl.BlockSpec((_BR, C), lambda i: (i, 0)),
            pl.BlockSpec((1, 1, _BR), lambda i: (i, 0, 0)),
        ],
        out_specs=pl.BlockSpec((_BR, C), lambda i: (i, 0)),
        out_shape=jax.ShapeDtypeStruct((B, C), input.dtype),
    )(input, mv3)
